# Initial kernel scaffold; baseline (speedup 1.0000x reference)
import jax
import jax.numpy as jnp
from jax.experimental import pallas as pl
from jax.experimental.pallas import tpu as pltpu
from jax.experimental.pallas import tpu_sc as plsc

_D = 32
_W = 128


def _gather_kernel(num_indices):
    mesh = plsc.VectorSubcoreMesh(core_axis_name="c", subcore_axis_name="s")

    @jax.jit
    def run(table, idx_flat):
        idx2d = idx_flat.reshape(1, num_indices)

        @pl.kernel(
            out_type=jax.ShapeDtypeStruct((num_indices, _D), jnp.float32),
            mesh=mesh,
        )
        def k(table_hbm, i_hbm, o_hbm):
            def body(i_vmem, o_vmem):
                pltpu.sync_copy(table_hbm.at[i_vmem.at[0]], o_vmem)

            pltpu.emit_pipeline(
                body,
                grid=(num_indices // _W,),
                in_specs=[pl.BlockSpec((1, _W), lambda i: (0, i))],
                out_specs=[pl.BlockSpec((_W, _D), lambda i: (i, 0))],
                core_axis_name=("c", "s"),
                dimension_semantics=(pltpu.PARALLEL,),
            )(i_hbm, o_hbm)

        return k(table, idx2d)

    return run


def kernel(input_text, embedding_table):
    batch, seq = input_text.shape
    n = batch * seq
    run = _gather_kernel(n)
    out = run(embedding_table, input_text.reshape(n))
    return out.reshape(batch, seq, _D)


# speedup vs baseline: 6.6387x; 6.6387x over previous
import jax
import jax.numpy as jnp
from jax.experimental import pallas as pl
from jax.experimental.pallas import tpu as pltpu
from jax.experimental.pallas import tpu_sc as plsc

_D = 32
_W = 128


def _gather_kernel(num_indices):
    mesh = plsc.VectorSubcoreMesh(core_axis_name="c", subcore_axis_name="s")

    @jax.jit
    def run(table, idx_flat):
        idx2d = idx_flat.reshape(1, num_indices)

        @pl.kernel(
            out_type=jax.ShapeDtypeStruct((num_indices, _D), jnp.float32),
            mesh=mesh,
            compiler_params=pltpu.CompilerParams(use_tc_tiling_on_sc=False),
        )
        def k(table_hbm, i_hbm, o_hbm):
            def body(i_vmem, o_vmem):
                pltpu.sync_copy(table_hbm.at[i_vmem.at[0]], o_vmem)

            pltpu.emit_pipeline(
                body,
                grid=(num_indices // _W,),
                in_specs=[pl.BlockSpec((1, _W), lambda i: (0, i))],
                out_specs=[pl.BlockSpec((_W, _D), lambda i: (i, 0))],
                core_axis_name=("c", "s"),
                dimension_semantics=(pltpu.PARALLEL,),
            )(i_hbm, o_hbm)

        return k(table, idx2d)

    return run


def kernel(input_text, embedding_table):
    batch, seq = input_text.shape
    n = batch * seq
    run = _gather_kernel(n)
    out = run(embedding_table, input_text.reshape(n))
    return out.reshape(batch, seq, _D)
